# Initial kernel scaffold; baseline (speedup 1.0000x reference)
import jax
import jax.numpy as jnp
from jax.experimental import pallas as pl


def kernel(logits):
    raise NotImplementedError("write your pallas kernel here")


# speedup vs baseline: 1.0352x; 1.0352x over previous
import functools

import jax
import jax.numpy as jnp
from jax.experimental import pallas as pl
from jax.experimental.pallas import tpu as pltpu

_BLOCK_C = 2048
_LANES = 128

_R0 = (13, 15, 26, 6)
_R1 = (17, 29, 16, 24)
_KS0 = 0
_KS1 = 42
_KS2 = _KS0 ^ _KS1 ^ 0x1BD11BDA

_TINY = 1.1754943508222875e-38
_SPAN = 1.0


def _rotl(x, r):
    return (x << jnp.uint32(r)) | (x >> jnp.uint32(32 - r))


def _threefry_rounds(x0, x1, rots):
    for r in rots:
        x0 = x0 + x1
        x1 = _rotl(x1, r)
        x1 = x1 ^ x0
    return x0, x1


def _threefry_bits(i):
    ks0 = jnp.uint32(_KS0)
    ks1 = jnp.uint32(_KS1)
    ks2 = jnp.uint32(_KS2)
    x0 = jnp.zeros_like(i) + ks0
    x1 = i + ks1
    x0, x1 = _threefry_rounds(x0, x1, _R0)
    x0 = x0 + ks1
    x1 = x1 + jnp.uint32(_KS2 + 1)
    x0, x1 = _threefry_rounds(x0, x1, _R1)
    x0 = x0 + ks2
    x1 = x1 + jnp.uint32(_KS0 + 2)
    x0, x1 = _threefry_rounds(x0, x1, _R0)
    x0 = x0 + ks0
    x1 = x1 + jnp.uint32(_KS1 + 3)
    x0, x1 = _threefry_rounds(x0, x1, _R1)
    x0 = x0 + ks1
    x1 = x1 + jnp.uint32(_KS2 + 4)
    x0, x1 = _threefry_rounds(x0, x1, _R0)
    x0 = x0 + ks2
    x1 = x1 + jnp.uint32(_KS0 + 5)
    return x0 ^ x1


def _sample_kernel(logits_ref, out_ref, max_ref, idx_ref, *, ncols, block_c,
                   nblocks):
    j = pl.program_id(0)
    nrows = logits_ref.shape[0]

    @pl.when(j == 0)
    def _init():
        max_ref[...] = jnp.full_like(max_ref, -jnp.inf)
        idx_ref[...] = jnp.zeros_like(idx_ref)

    row = jax.lax.broadcasted_iota(jnp.uint32, (nrows, block_c), 0)
    col = jax.lax.broadcasted_iota(jnp.int32, (nrows, block_c), 1)
    col = col + j * block_c
    i = row * jnp.uint32(ncols) + col.astype(jnp.uint32)

    bits = _threefry_bits(i)
    fb = (bits >> jnp.uint32(9)) | jnp.uint32(0x3F800000)
    f = jax.lax.bitcast_convert_type(fb, jnp.float32) - jnp.float32(1.0)
    tiny = jnp.float32(_TINY)
    u = jnp.maximum(tiny, f * jnp.float32(_SPAN) + tiny)
    g = -jnp.log(-jnp.log(u))
    phi = g + logits_ref[...]
    phi = jnp.where(col < ncols, phi, -jnp.inf)

    run_max = max_ref[...]
    run_idx = idx_ref[...]
    for k in range(block_c // _LANES):
        sl = slice(k * _LANES, (k + 1) * _LANES)
        chunk = phi[:, sl]
        cidx = col[:, sl]
        better = chunk > run_max
        run_max = jnp.where(better, chunk, run_max)
        run_idx = jnp.where(better, cidx, run_idx)
    max_ref[...] = run_max
    idx_ref[...] = run_idx

    @pl.when(j == nblocks - 1)
    def _finish():
        m = max_ref[...]
        ix = idx_ref[...]
        row_max = jnp.max(m, axis=1, keepdims=True)
        cand = jnp.where(m == row_max, ix, jnp.int32(0x7FFFFFFF))
        out_ref[...] = jnp.broadcast_to(jnp.min(cand, axis=1, keepdims=True),
                                        out_ref.shape)


@jax.jit
def kernel(logits):
    nrows, ncols = logits.shape
    block_c = _BLOCK_C
    nblocks = pl.cdiv(ncols, block_c)
    out = pl.pallas_call(
        functools.partial(_sample_kernel, ncols=ncols, block_c=block_c,
                          nblocks=nblocks),
        grid=(nblocks,),
        in_specs=[pl.BlockSpec((nrows, block_c), lambda j: (0, j))],
        out_specs=pl.BlockSpec((nrows, _LANES), lambda j: (0, 0)),
        out_shape=jax.ShapeDtypeStruct((nrows, _LANES), jnp.int32),
        scratch_shapes=[
            pltpu.VMEM((nrows, _LANES), jnp.float32),
            pltpu.VMEM((nrows, _LANES), jnp.int32),
        ],
    )(logits)
    return out[:, :1]
